# baseline (speedup 1.0000x reference)
import jax
import jax.numpy as jnp
from jax.experimental import pallas as pl
from jax.experimental.pallas import tpu as pltpu

_ROWS = 2048


def _fold_body(w1_ref, b1_ref, w2_ref, b2_ref, wc_ref, bc_ref):
    wc = jnp.dot(w1_ref[...], w2_ref[...], preferred_element_type=jnp.float32)
    wc_ref[...] = wc.astype(jnp.bfloat16)
    bc_ref[...] = jnp.dot(b1_ref[...], w2_ref[...],
                          preferred_element_type=jnp.float32) + b2_ref[...]


def _mm_body(x_ref, wc_ref, bc_ref, o_ref):
    xb = x_ref[...].astype(jnp.bfloat16)
    o = jnp.dot(xb, wc_ref[...], preferred_element_type=jnp.float32)
    o_ref[...] = o + bc_ref[...]


def kernel(x, graph_x, edge_index, W_ec, b_ec, W1, b1, W2, b2):
    n, d_in = x.shape
    hid = W1.shape[1]
    d_out = W2.shape[1]
    b1r = b1.reshape(1, hid)
    b2r = b2.reshape(1, d_out)

    wc, bc = pl.pallas_call(
        _fold_body,
        out_shape=(
            jax.ShapeDtypeStruct((d_in, d_out), jnp.bfloat16),
            jax.ShapeDtypeStruct((1, d_out), jnp.float32),
        ),
    )(W1, b1r, W2, b2r)

    grid = (pl.cdiv(n, _ROWS),)
    out = pl.pallas_call(
        _mm_body,
        grid=grid,
        in_specs=[
            pl.BlockSpec((_ROWS, d_in), lambda i: (i, 0)),
            pl.BlockSpec((d_in, d_out), lambda i: (0, 0)),
            pl.BlockSpec((1, d_out), lambda i: (0, 0)),
        ],
        out_specs=pl.BlockSpec((_ROWS, d_out), lambda i: (i, 0)),
        out_shape=jax.ShapeDtypeStruct((n, d_out), x.dtype),
        compiler_params=pltpu.CompilerParams(
            dimension_semantics=("parallel",),
        ),
    )(x, wc, bc)
    return out


# speedup vs baseline: 1.5862x; 1.5862x over previous
import jax
import jax.numpy as jnp
from jax.experimental import pallas as pl
from jax.experimental.pallas import tpu as pltpu

_ROWS = 8192


def _fold_body(w1_ref, b1_ref, w2_ref, b2_ref, wc_ref, bc_ref):
    wc = jnp.dot(w1_ref[...], w2_ref[...], preferred_element_type=jnp.float32)
    wc_ref[...] = wc.astype(jnp.bfloat16)
    bc_ref[...] = jnp.dot(b1_ref[...], w2_ref[...],
                          preferred_element_type=jnp.float32) + b2_ref[...]


def _mm_body(x_ref, wc_ref, bc_ref, o_ref):
    xb = x_ref[...].astype(jnp.bfloat16)
    o = jnp.dot(xb, wc_ref[...], preferred_element_type=jnp.float32)
    o_ref[...] = o + bc_ref[...]


def kernel(x, graph_x, edge_index, W_ec, b_ec, W1, b1, W2, b2):
    n, d_in = x.shape
    hid = W1.shape[1]
    d_out = W2.shape[1]
    b1r = b1.reshape(1, hid)
    b2r = b2.reshape(1, d_out)

    wc, bc = pl.pallas_call(
        _fold_body,
        out_shape=(
            jax.ShapeDtypeStruct((d_in, d_out), jnp.bfloat16),
            jax.ShapeDtypeStruct((1, d_out), jnp.float32),
        ),
    )(W1, b1r, W2, b2r)

    grid = (pl.cdiv(n, _ROWS),)
    out = pl.pallas_call(
        _mm_body,
        grid=grid,
        in_specs=[
            pl.BlockSpec((_ROWS, d_in), lambda i: (i, 0)),
            pl.BlockSpec((d_in, d_out), lambda i: (0, 0)),
            pl.BlockSpec((1, d_out), lambda i: (0, 0)),
        ],
        out_specs=pl.BlockSpec((_ROWS, d_out), lambda i: (i, 0)),
        out_shape=jax.ShapeDtypeStruct((n, d_out), x.dtype),
        compiler_params=pltpu.CompilerParams(
            dimension_semantics=("parallel",),
        ),
    )(x, wc, bc)
    return out


# speedup vs baseline: 1.6553x; 1.0435x over previous
import jax
import jax.numpy as jnp
from jax.experimental import pallas as pl
from jax.experimental.pallas import tpu as pltpu

_ROWS = 20000


def _fold_body(w1_ref, b1_ref, w2_ref, b2_ref, wc_ref, bc_ref):
    wc = jnp.dot(w1_ref[...], w2_ref[...], preferred_element_type=jnp.float32)
    wc_ref[...] = wc.astype(jnp.bfloat16)
    bc_ref[...] = jnp.dot(b1_ref[...], w2_ref[...],
                          preferred_element_type=jnp.float32) + b2_ref[...]


def _mm_body(x_ref, wc_ref, bc_ref, o_ref):
    xb = x_ref[...].astype(jnp.bfloat16)
    o = jnp.dot(xb, wc_ref[...], preferred_element_type=jnp.float32)
    o_ref[...] = o + bc_ref[...]


def kernel(x, graph_x, edge_index, W_ec, b_ec, W1, b1, W2, b2):
    n, d_in = x.shape
    hid = W1.shape[1]
    d_out = W2.shape[1]
    b1r = b1.reshape(1, hid)
    b2r = b2.reshape(1, d_out)

    wc, bc = pl.pallas_call(
        _fold_body,
        out_shape=(
            jax.ShapeDtypeStruct((d_in, d_out), jnp.bfloat16),
            jax.ShapeDtypeStruct((1, d_out), jnp.float32),
        ),
    )(W1, b1r, W2, b2r)

    grid = (pl.cdiv(n, _ROWS),)
    out = pl.pallas_call(
        _mm_body,
        grid=grid,
        in_specs=[
            pl.BlockSpec((_ROWS, d_in), lambda i: (i, 0)),
            pl.BlockSpec((d_in, d_out), lambda i: (0, 0)),
            pl.BlockSpec((1, d_out), lambda i: (0, 0)),
        ],
        out_specs=pl.BlockSpec((_ROWS, d_out), lambda i: (i, 0)),
        out_shape=jax.ShapeDtypeStruct((n, d_out), x.dtype),
        compiler_params=pltpu.CompilerParams(
            dimension_semantics=("parallel",),
        ),
    )(x, wc, bc)
    return out


# speedup vs baseline: 1.7264x; 1.0429x over previous
import jax
import jax.numpy as jnp
from jax.experimental import pallas as pl
from jax.experimental.pallas import tpu as pltpu

_ROWS = 20000


def _mlp_body(x_ref, w1_ref, b1_ref, w2_ref, b2_ref, o_ref, wc_ref, bc_ref):
    @pl.when(pl.program_id(0) == 0)
    def _fold_weights():
        wc = jnp.dot(w1_ref[...], w2_ref[...], preferred_element_type=jnp.float32)
        wc_ref[...] = wc.astype(jnp.bfloat16)
        bc_ref[...] = jnp.dot(b1_ref[...], w2_ref[...],
                              preferred_element_type=jnp.float32) + b2_ref[...]

    xb = x_ref[...].astype(jnp.bfloat16)
    o = jnp.dot(xb, wc_ref[...], preferred_element_type=jnp.float32)
    o_ref[...] = o + bc_ref[...]


def kernel(x, graph_x, edge_index, W_ec, b_ec, W1, b1, W2, b2):
    n, d_in = x.shape
    hid = W1.shape[1]
    d_out = W2.shape[1]
    b1r = b1.reshape(1, hid)
    b2r = b2.reshape(1, d_out)
    grid = (pl.cdiv(n, _ROWS),)
    out = pl.pallas_call(
        _mlp_body,
        grid=grid,
        in_specs=[
            pl.BlockSpec((_ROWS, d_in), lambda i: (i, 0)),
            pl.BlockSpec((d_in, hid), lambda i: (0, 0)),
            pl.BlockSpec((1, hid), lambda i: (0, 0)),
            pl.BlockSpec((hid, d_out), lambda i: (0, 0)),
            pl.BlockSpec((1, d_out), lambda i: (0, 0)),
        ],
        out_specs=pl.BlockSpec((_ROWS, d_out), lambda i: (i, 0)),
        out_shape=jax.ShapeDtypeStruct((n, d_out), x.dtype),
        scratch_shapes=[
            pltpu.VMEM((d_in, d_out), jnp.bfloat16),
            pltpu.VMEM((1, d_out), jnp.float32),
        ],
    )(x, W1, b1r, W2, b2r)
    return out
